# Initial kernel scaffold; baseline (speedup 1.0000x reference)
import jax
import jax.numpy as jnp
from jax.experimental import pallas as pl


def kernel(users, items, user_emb, item_emb, edge_src, edge_dst, edge_val):
    raise NotImplementedError("write your pallas kernel here")


# speedup vs baseline: 13.1451x; 13.1451x over previous
import functools

import jax
import jax.numpy as jnp
from jax import lax
from jax.experimental import pallas as pl
from jax.experimental.pallas import tpu as pltpu
from jax.experimental.pallas import tpu_sc as plsc

_N_USERS = 25000
_N = 50000
_D = 32
_E = 1600000
_NPAD = 51200
_EPAD = 1605632
_GPW = 392
_K = 8
_CHUNKS = _GPW // _K
_C = _K * 128
_RPT = _NPAD // 16

_MESH = plsc.VectorSubcoreMesh(core_axis_name="c", subcore_axis_name="s")


@functools.partial(
    pl.kernel,
    out_type=[
        jax.ShapeDtypeStruct((_NPAD, _D), jnp.float32),
        jax.ShapeDtypeStruct((_NPAD, _D), jnp.float32),
    ],
    mesh=_MESH,
    compiler_params=pltpu.CompilerParams(use_tc_tiling_on_sc=False),
    scratch_types=[
        pltpu.VMEM((_K, 128), jnp.int32),
        pltpu.VMEM((_K, 128), jnp.int32),
        pltpu.VMEM((_C,), jnp.float32),
        pltpu.VMEM((128, _D), jnp.float32),
        pltpu.VMEM_SHARED((_NPAD, _D), jnp.float32),
        pltpu.SemaphoreType.DMA,
    ],
)
def _prop(table, srcg, dstg, valf, p0, p1, src_v, dst_v, val_v, rows_v, acc, sem):
    cid = lax.axis_index("c")
    sid = lax.axis_index("s")
    wid = sid * 2 + cid

    def _zrow(i, carry):
        z = jnp.zeros((16,), jnp.float32)
        rows_v[i, pl.ds(0, 16)] = z
        rows_v[i, pl.ds(16, 16)] = z
        return carry

    lax.fori_loop(0, 128, _zrow, 0)

    def _zacc(h, carry):
        pltpu.sync_copy(rows_v, acc.at[pl.ds(sid * _RPT + h * 128, 128)])
        return carry

    lax.fori_loop(0, _RPT // 128, _zacc, 0)
    plsc.subcore_barrier()

    def _chunk(c, carry):
        row0 = wid * _GPW + c * _K
        pltpu.sync_copy(srcg.at[pl.ds(row0, _K)], src_v)
        pltpu.sync_copy(dstg.at[pl.ds(row0, _K)], dst_v)
        pltpu.sync_copy(valf.at[pl.ds(row0 * 128, _C)], val_v)

        def _grp(j, carry2):
            pltpu.async_copy(table.at[src_v.at[j]], rows_v, sem).wait()

            def _scale(g, carry3):
                v16 = val_v[pl.ds(j * 128 + g * 16, 16)]
                for i in range(16):
                    e = g * 16 + i
                    v = v16[i]
                    rows_v[e, pl.ds(0, 16)] = rows_v[e, pl.ds(0, 16)] * v
                    rows_v[e, pl.ds(16, 16)] = rows_v[e, pl.ds(16, 16)] * v
                return carry3

            lax.fori_loop(0, 8, _scale, 0)

            pltpu.sync_copy(rows_v, acc.at[dst_v.at[j]], add=True)
            return carry2

        lax.fori_loop(0, _K, _grp, 0)
        return carry

    lax.fori_loop(0, _CHUNKS, _chunk, 0)

    plsc.subcore_barrier()
    r0 = sid * _RPT

    @pl.when(cid == 0)
    def _():
        pltpu.sync_copy(acc.at[pl.ds(r0, _RPT)], p0.at[pl.ds(r0, _RPT)])

    @pl.when(cid == 1)
    def _():
        pltpu.sync_copy(acc.at[pl.ds(r0, _RPT)], p1.at[pl.ds(r0, _RPT)])


def _merge_body(p0_ref, p1_ref, s_ref, t_out, s_out):
    t = p0_ref[...] + p1_ref[...]
    t_out[...] = t
    s_out[...] = s_ref[...] + t


def _merge(p0, p1, s):
    rows = _NPAD * _D // 128
    blk = rows // 8
    f = pl.pallas_call(
        _merge_body,
        out_shape=[jax.ShapeDtypeStruct((rows, 128), jnp.float32)] * 2,
        grid=(8,),
        in_specs=[pl.BlockSpec((blk, 128), lambda i: (i, 0))] * 3,
        out_specs=[pl.BlockSpec((blk, 128), lambda i: (i, 0))] * 2,
    )
    t, s2 = f(
        p0.reshape(rows, 128), p1.reshape(rows, 128), s.reshape(rows, 128)
    )
    return t.reshape(_NPAD, _D), s2.reshape(_NPAD, _D)


@functools.partial(
    pl.kernel,
    out_type=[
        jax.ShapeDtypeStruct((4096, _D), jnp.float32),
        jax.ShapeDtypeStruct((4096, _D), jnp.float32),
    ],
    mesh=_MESH,
    compiler_params=pltpu.CompilerParams(use_tc_tiling_on_sc=False),
    scratch_types=[
        pltpu.VMEM((128,), jnp.int32),
        pltpu.VMEM((128,), jnp.int32),
        pltpu.VMEM((128, _D), jnp.float32),
        pltpu.VMEM((128, _D), jnp.float32),
        pltpu.SemaphoreType.DMA,
    ],
)
def _gather2(sum_t, uid, gid, ur_o, ir_o, uid_v, gid_v, ur_v, ir_v, sem):
    cid = lax.axis_index("c")
    sid = lax.axis_index("s")
    base = (sid * 2 + cid) * 128
    pltpu.sync_copy(uid.at[pl.ds(base, 128)], uid_v)
    pltpu.sync_copy(gid.at[pl.ds(base, 128)], gid_v)
    a = pltpu.async_copy(sum_t.at[uid_v], ur_v, sem)
    b = pltpu.async_copy(sum_t.at[gid_v], ir_v, sem)
    a.wait()
    b.wait()
    pltpu.sync_copy(ur_v, ur_o.at[pl.ds(base, 128)])
    pltpu.sync_copy(ir_v, ir_o.at[pl.ds(base, 128)])


def _dot_body(u_ref, i_ref, o_ref):
    o_ref[...] = jnp.sum(u_ref[...] * i_ref[...], axis=1) * jnp.float32(1.0 / 16.0)


def kernel(users, items, user_emb, item_emb, edge_src, edge_dst, edge_val):
    table0 = (
        jnp.zeros((_NPAD, _D), jnp.float32)
        .at[:_N]
        .set(jnp.concatenate([user_emb, item_emb], axis=0))
    )
    pad = _EPAD - _E
    srcg = jnp.concatenate([edge_src, jnp.zeros((pad,), jnp.int32)]).reshape(
        _EPAD // 128, 128
    )
    dstg = jnp.concatenate([edge_dst, jnp.zeros((pad,), jnp.int32)]).reshape(
        _EPAD // 128, 128
    )
    valf = jnp.concatenate([edge_val, jnp.zeros((pad,), jnp.float32)])
    table = table0
    s = table0
    for _ in range(3):
        p0, p1 = _prop(table, srcg, dstg, valf)
        table, s = _merge(p0, p1, s)
    ur, ir = _gather2(s, users, items + jnp.int32(_N_USERS))
    return pl.pallas_call(
        _dot_body, out_shape=jax.ShapeDtypeStruct((4096,), jnp.float32)
    )(ur, ir)


# speedup vs baseline: 25.7737x; 1.9607x over previous
import functools

import jax
import jax.numpy as jnp
from jax import lax
from jax.experimental import pallas as pl
from jax.experimental.pallas import tpu as pltpu
from jax.experimental.pallas import tpu_sc as plsc

_N_USERS = 25000
_N = 50000
_D = 32
_E = 1600000
_NPAD = 51200
_EPAD = 1605632
_GPW = 392
_K = 8
_CHUNKS = _GPW // _K
_C = _K * 128
_RPT = _NPAD // 16

_MESH = plsc.VectorSubcoreMesh(core_axis_name="c", subcore_axis_name="s")


@functools.partial(
    pl.kernel,
    out_type=[
        jax.ShapeDtypeStruct((_NPAD, _D), jnp.float32),
        jax.ShapeDtypeStruct((_NPAD, _D), jnp.float32),
    ],
    mesh=_MESH,
    compiler_params=pltpu.CompilerParams(use_tc_tiling_on_sc=False),
    scratch_types=[
        pltpu.VMEM((2, _K, 128), jnp.int32),
        pltpu.VMEM((2, _K, 128), jnp.int32),
        pltpu.VMEM((2, _C), jnp.float32),
        pltpu.VMEM((4, 128, _D), jnp.float32),
        pltpu.VMEM_SHARED((_NPAD, _D), jnp.float32),
    ]
    + [pltpu.SemaphoreType.DMA] * 10,
)
def _prop(
    table, srcg, dstg, valf, p0, p1, src_v, dst_v, val_v, rows_v, acc,
    g0, g1, g2, g3, s0, s1, s2, s3, stsem, zsem,
):
    cid = lax.axis_index("c")
    sid = lax.axis_index("s")
    wid = sid * 2 + cid
    gsem = [g0, g1, g2, g3]
    ssem = [s0, s1, s2, s3]

    def _scat_wait(b):
        pltpu.make_async_copy(
            table.at[pl.ds(0, 128)], rows_v.at[b], ssem[b]
        ).wait()

    def _zrow(i, carry):
        z = jnp.zeros((16,), jnp.float32)
        rows_v[0, i, pl.ds(0, 16)] = z
        rows_v[0, i, pl.ds(16, 16)] = z
        return carry

    lax.fori_loop(0, 128, _zrow, 0)
    zdescs = [
        pltpu.async_copy(
            rows_v.at[0], acc.at[pl.ds(sid * _RPT + h * 128, 128)], zsem
        )
        for h in range(_RPT // 128)
    ]
    for dsc in zdescs:
        dsc.wait()
    plsc.subcore_barrier()

    row00 = wid * _GPW
    pltpu.async_copy(srcg.at[pl.ds(row00, _K)], src_v.at[0], stsem)
    pltpu.async_copy(dstg.at[pl.ds(row00, _K)], dst_v.at[0], stsem)
    pltpu.async_copy(valf.at[pl.ds(row00 * 128, _C)], val_v.at[0], stsem)

    def _scale(b, voff, val_p):
        def _sg(g, carry3):
            v16 = val_p[pl.ds(voff + g * 16, 16)]
            for i in range(16):
                e = g * 16 + i
                v = v16[i]
                rows_v[b, e, pl.ds(0, 16)] = rows_v[b, e, pl.ds(0, 16)] * v
                rows_v[b, e, pl.ds(16, 16)] = rows_v[b, e, pl.ds(16, 16)] * v
            return carry3

        lax.fori_loop(0, 8, _sg, 0)

    def _chunk(c, carry):
        p = lax.rem(c, 2)
        pltpu.make_async_copy(srcg.at[pl.ds(0, _K)], src_v.at[p], stsem).wait()
        pltpu.make_async_copy(dstg.at[pl.ds(0, _K)], dst_v.at[p], stsem).wait()
        pltpu.make_async_copy(valf.at[pl.ds(0, _C)], val_v.at[p], stsem).wait()
        @pl.when(c < _CHUNKS - 1)
        def _():
            cn = c + 1
            pn = lax.rem(cn, 2)
            rown = wid * _GPW + cn * _K
            pltpu.async_copy(srcg.at[pl.ds(rown, _K)], src_v.at[pn], stsem)
            pltpu.async_copy(dstg.at[pl.ds(rown, _K)], dst_v.at[pn], stsem)
            pltpu.async_copy(valf.at[pl.ds(rown * 128, _C)], val_v.at[pn], stsem)

        @pl.when(c > 0)
        def _():
            _scat_wait(0)

        gdescs = [None] * _K
        sdescs = [None] * _K
        gdescs[0] = pltpu.async_copy(
            table.at[src_v.at[p, 0]], rows_v.at[0], gsem[0]
        )
        for j in range(_K):
            b = j % 4
            if j < _K - 1:
                b1 = (j + 1) % 4
                if j + 1 < 4:

                    @pl.when(c > 0)
                    def _():
                        _scat_wait(b1)

                else:
                    sdescs[j + 1 - 4].wait()
                gdescs[j + 1] = pltpu.async_copy(
                    table.at[src_v.at[p, j + 1]], rows_v.at[b1], gsem[b1]
                )
            gdescs[j].wait()
            _scale(b, j * 128, val_v.at[p])
            sdescs[j] = pltpu.async_copy(
                rows_v.at[b], acc.at[dst_v.at[p, j]], ssem[b], add=True
            )
        return carry

    lax.fori_loop(0, _CHUNKS, _chunk, 0)
    for b in range(4):
        _scat_wait(b)

    plsc.subcore_barrier()
    r0 = sid * _RPT

    @pl.when(cid == 0)
    def _():
        pltpu.sync_copy(acc.at[pl.ds(r0, _RPT)], p0.at[pl.ds(r0, _RPT)])

    @pl.when(cid == 1)
    def _():
        pltpu.sync_copy(acc.at[pl.ds(r0, _RPT)], p1.at[pl.ds(r0, _RPT)])


def _merge_body(p0_ref, p1_ref, s_ref, t_out, s_out):
    t = p0_ref[...] + p1_ref[...]
    t_out[...] = t
    s_out[...] = s_ref[...] + t


def _merge(p0, p1, s):
    rows = _NPAD * _D // 128
    blk = rows // 8
    f = pl.pallas_call(
        _merge_body,
        out_shape=[jax.ShapeDtypeStruct((rows, 128), jnp.float32)] * 2,
        grid=(8,),
        in_specs=[pl.BlockSpec((blk, 128), lambda i: (i, 0))] * 3,
        out_specs=[pl.BlockSpec((blk, 128), lambda i: (i, 0))] * 2,
    )
    t, s2 = f(
        p0.reshape(rows, 128), p1.reshape(rows, 128), s.reshape(rows, 128)
    )
    return t.reshape(_NPAD, _D), s2.reshape(_NPAD, _D)


@functools.partial(
    pl.kernel,
    out_type=[
        jax.ShapeDtypeStruct((4096, _D), jnp.float32),
        jax.ShapeDtypeStruct((4096, _D), jnp.float32),
    ],
    mesh=_MESH,
    compiler_params=pltpu.CompilerParams(use_tc_tiling_on_sc=False),
    scratch_types=[
        pltpu.VMEM((128,), jnp.int32),
        pltpu.VMEM((128,), jnp.int32),
        pltpu.VMEM((128, _D), jnp.float32),
        pltpu.VMEM((128, _D), jnp.float32),
        pltpu.SemaphoreType.DMA,
    ],
)
def _gather2(sum_t, uid, gid, ur_o, ir_o, uid_v, gid_v, ur_v, ir_v, sem):
    cid = lax.axis_index("c")
    sid = lax.axis_index("s")
    base = (sid * 2 + cid) * 128
    pltpu.sync_copy(uid.at[pl.ds(base, 128)], uid_v)
    pltpu.sync_copy(gid.at[pl.ds(base, 128)], gid_v)
    a = pltpu.async_copy(sum_t.at[uid_v], ur_v, sem)
    b = pltpu.async_copy(sum_t.at[gid_v], ir_v, sem)
    a.wait()
    b.wait()
    pltpu.sync_copy(ur_v, ur_o.at[pl.ds(base, 128)])
    pltpu.sync_copy(ir_v, ir_o.at[pl.ds(base, 128)])


def _dot_body(u_ref, i_ref, o_ref):
    o_ref[...] = jnp.sum(u_ref[...] * i_ref[...], axis=1) * jnp.float32(1.0 / 16.0)


def kernel(users, items, user_emb, item_emb, edge_src, edge_dst, edge_val):
    table0 = (
        jnp.zeros((_NPAD, _D), jnp.float32)
        .at[:_N]
        .set(jnp.concatenate([user_emb, item_emb], axis=0))
    )
    pad = _EPAD - _E
    srcg = jnp.concatenate([edge_src, jnp.zeros((pad,), jnp.int32)]).reshape(
        _EPAD // 128, 128
    )
    dstg = jnp.concatenate([edge_dst, jnp.zeros((pad,), jnp.int32)]).reshape(
        _EPAD // 128, 128
    )
    valf = jnp.concatenate([edge_val, jnp.zeros((pad,), jnp.float32)])
    table = table0
    s = table0
    for _ in range(3):
        p0, p1 = _prop(table, srcg, dstg, valf)
        table, s = _merge(p0, p1, s)
    ur, ir = _gather2(s, users, items + jnp.int32(_N_USERS))
    return pl.pallas_call(
        _dot_body, out_shape=jax.ShapeDtypeStruct((4096,), jnp.float32)
    )(ur, ir)


# speedup vs baseline: 28.6244x; 1.1106x over previous
import functools

import jax
import jax.numpy as jnp
from jax import lax
from jax.experimental import pallas as pl
from jax.experimental.pallas import tpu as pltpu
from jax.experimental.pallas import tpu_sc as plsc

_N_USERS = 25000
_N = 50000
_D = 32
_E = 1600000
_NPAD = 51200
_EPAD = 1605632
_GPW = 392
_K = 8
_CHUNKS = _GPW // _K
_C = _K * 128
_RPT = _NPAD // 16

_MESH = plsc.VectorSubcoreMesh(core_axis_name="c", subcore_axis_name="s")


@functools.partial(
    pl.kernel,
    out_type=[
        jax.ShapeDtypeStruct((_NPAD, _D), jnp.float32),
        jax.ShapeDtypeStruct((_NPAD, _D), jnp.float32),
    ],
    mesh=_MESH,
    compiler_params=pltpu.CompilerParams(use_tc_tiling_on_sc=False),
    scratch_types=[
        pltpu.VMEM((2, _K, 128), jnp.int32),
        pltpu.VMEM((2, _K, 128), jnp.int32),
        pltpu.VMEM((2, _C), jnp.float32),
        pltpu.VMEM((4, 128, _D), jnp.float32),
        pltpu.VMEM_SHARED((_NPAD, _D), jnp.float32),
    ]
    + [pltpu.SemaphoreType.DMA] * 10,
)
def _prop(
    table, srcg, dstg, valf, p0, p1, src_v, dst_v, val_v, rows_v, acc,
    g0, g1, g2, g3, s0, s1, s2, s3, stsem, zsem,
):
    cid = lax.axis_index("c")
    sid = lax.axis_index("s")
    wid = sid * 2 + cid
    gsem = [g0, g1, g2, g3]
    ssem = [s0, s1, s2, s3]

    def _scat_wait(b):
        pltpu.make_async_copy(
            table.at[pl.ds(0, 128)], rows_v.at[b], ssem[b]
        ).wait()

    def _zrow(i, carry):
        z = jnp.zeros((16,), jnp.float32)
        rows_v[0, i, pl.ds(0, 16)] = z
        rows_v[0, i, pl.ds(16, 16)] = z
        return carry

    lax.fori_loop(0, 128, _zrow, 0)
    zdescs = [
        pltpu.async_copy(
            rows_v.at[0], acc.at[pl.ds(sid * _RPT + h * 128, 128)], zsem
        )
        for h in range(_RPT // 128)
    ]
    for dsc in zdescs:
        dsc.wait()
    plsc.subcore_barrier()

    row00 = wid * _GPW
    pltpu.async_copy(srcg.at[pl.ds(row00, _K)], src_v.at[0], stsem)
    pltpu.async_copy(dstg.at[pl.ds(row00, _K)], dst_v.at[0], stsem)
    pltpu.async_copy(valf.at[pl.ds(row00 * 128, _C)], val_v.at[0], stsem)

    def _scale(b, voff, val_p):
        def _sg(g, carry3):
            v16 = val_p[pl.ds(voff + g * 16, 16)]
            for i in range(16):
                e = g * 16 + i
                sp = lax.gather(
                    v16,
                    jnp.full((16, 1), i, jnp.int32),
                    lax.GatherDimensionNumbers(
                        offset_dims=(),
                        collapsed_slice_dims=(0,),
                        start_index_map=(0,),
                    ),
                    (1,),
                    mode=lax.GatherScatterMode.PROMISE_IN_BOUNDS,
                )
                rows_v[b, e, pl.ds(0, 16)] = rows_v[b, e, pl.ds(0, 16)] * sp
                rows_v[b, e, pl.ds(16, 16)] = rows_v[b, e, pl.ds(16, 16)] * sp
            return carry3

        lax.fori_loop(0, 8, _sg, 0)

    def _chunk(c, carry):
        p = lax.rem(c, 2)
        pltpu.make_async_copy(srcg.at[pl.ds(0, _K)], src_v.at[p], stsem).wait()
        pltpu.make_async_copy(dstg.at[pl.ds(0, _K)], dst_v.at[p], stsem).wait()
        pltpu.make_async_copy(valf.at[pl.ds(0, _C)], val_v.at[p], stsem).wait()
        @pl.when(c < _CHUNKS - 1)
        def _():
            cn = c + 1
            pn = lax.rem(cn, 2)
            rown = wid * _GPW + cn * _K
            pltpu.async_copy(srcg.at[pl.ds(rown, _K)], src_v.at[pn], stsem)
            pltpu.async_copy(dstg.at[pl.ds(rown, _K)], dst_v.at[pn], stsem)
            pltpu.async_copy(valf.at[pl.ds(rown * 128, _C)], val_v.at[pn], stsem)

        @pl.when(c > 0)
        def _():
            _scat_wait(0)
            _scat_wait(1)

        gdescs = [None] * _K
        sdescs = [None] * _K
        for j in range(2):
            gdescs[j] = pltpu.async_copy(
                table.at[src_v.at[p, j]], rows_v.at[j], gsem[j]
            )
        for j in range(_K):
            b = j % 4
            if j < _K - 2:
                b2 = (j + 2) % 4
                if j + 2 < 4:

                    @pl.when(c > 0)
                    def _():
                        _scat_wait(b2)

                else:
                    sdescs[j - 2].wait()
                gdescs[j + 2] = pltpu.async_copy(
                    table.at[src_v.at[p, j + 2]], rows_v.at[b2], gsem[b2]
                )
            gdescs[j].wait()
            _scale(b, j * 128, val_v.at[p])
            sdescs[j] = pltpu.async_copy(
                rows_v.at[b], acc.at[dst_v.at[p, j]], ssem[b], add=True
            )
        return carry

    lax.fori_loop(0, _CHUNKS, _chunk, 0)
    for b in range(4):
        _scat_wait(b)

    plsc.subcore_barrier()
    r0 = sid * _RPT

    @pl.when(cid == 0)
    def _():
        pltpu.sync_copy(acc.at[pl.ds(r0, _RPT)], p0.at[pl.ds(r0, _RPT)])

    @pl.when(cid == 1)
    def _():
        pltpu.sync_copy(acc.at[pl.ds(r0, _RPT)], p1.at[pl.ds(r0, _RPT)])


def _merge_body(p0_ref, p1_ref, s_ref, t_out, s_out):
    t = p0_ref[...] + p1_ref[...]
    t_out[...] = t
    s_out[...] = s_ref[...] + t


def _merge(p0, p1, s):
    rows = _NPAD * _D // 128
    blk = rows // 8
    f = pl.pallas_call(
        _merge_body,
        out_shape=[jax.ShapeDtypeStruct((rows, 128), jnp.float32)] * 2,
        grid=(8,),
        in_specs=[pl.BlockSpec((blk, 128), lambda i: (i, 0))] * 3,
        out_specs=[pl.BlockSpec((blk, 128), lambda i: (i, 0))] * 2,
    )
    t, s2 = f(
        p0.reshape(rows, 128), p1.reshape(rows, 128), s.reshape(rows, 128)
    )
    return t.reshape(_NPAD, _D), s2.reshape(_NPAD, _D)


@functools.partial(
    pl.kernel,
    out_type=[
        jax.ShapeDtypeStruct((4096, _D), jnp.float32),
        jax.ShapeDtypeStruct((4096, _D), jnp.float32),
    ],
    mesh=_MESH,
    compiler_params=pltpu.CompilerParams(use_tc_tiling_on_sc=False),
    scratch_types=[
        pltpu.VMEM((128,), jnp.int32),
        pltpu.VMEM((128,), jnp.int32),
        pltpu.VMEM((128, _D), jnp.float32),
        pltpu.VMEM((128, _D), jnp.float32),
        pltpu.SemaphoreType.DMA,
    ],
)
def _gather2(sum_t, uid, gid, ur_o, ir_o, uid_v, gid_v, ur_v, ir_v, sem):
    cid = lax.axis_index("c")
    sid = lax.axis_index("s")
    base = (sid * 2 + cid) * 128
    pltpu.sync_copy(uid.at[pl.ds(base, 128)], uid_v)
    pltpu.sync_copy(gid.at[pl.ds(base, 128)], gid_v)
    a = pltpu.async_copy(sum_t.at[uid_v], ur_v, sem)
    b = pltpu.async_copy(sum_t.at[gid_v], ir_v, sem)
    a.wait()
    b.wait()
    pltpu.sync_copy(ur_v, ur_o.at[pl.ds(base, 128)])
    pltpu.sync_copy(ir_v, ir_o.at[pl.ds(base, 128)])


def _dot_body(u_ref, i_ref, o_ref):
    o_ref[...] = jnp.sum(u_ref[...] * i_ref[...], axis=1) * jnp.float32(1.0 / 16.0)


def kernel(users, items, user_emb, item_emb, edge_src, edge_dst, edge_val):
    table0 = (
        jnp.zeros((_NPAD, _D), jnp.float32)
        .at[:_N]
        .set(jnp.concatenate([user_emb, item_emb], axis=0))
    )
    pad = _EPAD - _E
    srcg = jnp.concatenate([edge_src, jnp.zeros((pad,), jnp.int32)]).reshape(
        _EPAD // 128, 128
    )
    dstg = jnp.concatenate([edge_dst, jnp.zeros((pad,), jnp.int32)]).reshape(
        _EPAD // 128, 128
    )
    valf = jnp.concatenate([edge_val, jnp.zeros((pad,), jnp.float32)])
    table = table0
    s = table0
    for _ in range(3):
        p0, p1 = _prop(table, srcg, dstg, valf)
        table, s = _merge(p0, p1, s)
    ur, ir = _gather2(s, users, items + jnp.int32(_N_USERS))
    return pl.pallas_call(
        _dot_body, out_shape=jax.ShapeDtypeStruct((4096,), jnp.float32)
    )(ur, ir)


# speedup vs baseline: 28.8088x; 1.0064x over previous
import functools

import jax
import jax.numpy as jnp
from jax import lax
from jax.experimental import pallas as pl
from jax.experimental.pallas import tpu as pltpu
from jax.experimental.pallas import tpu_sc as plsc

_N_USERS = 25000
_N = 50000
_D = 32
_E = 1600000
_NPAD = 51200
_EPAD = 1605632
_GPW = 392
_K = 8
_CHUNKS = _GPW // _K
_C = _K * 128
_RPT = _NPAD // 16

_MESH = plsc.VectorSubcoreMesh(core_axis_name="c", subcore_axis_name="s")


@functools.partial(
    pl.kernel,
    out_type=[
        jax.ShapeDtypeStruct((_NPAD, _D), jnp.float32),
        jax.ShapeDtypeStruct((_NPAD, _D), jnp.float32),
    ],
    mesh=_MESH,
    compiler_params=pltpu.CompilerParams(use_tc_tiling_on_sc=False),
    scratch_types=[
        pltpu.VMEM((2, _K, 128), jnp.int32),
        pltpu.VMEM((2, _K, 128), jnp.int32),
        pltpu.VMEM((2, _C), jnp.float32),
        pltpu.VMEM((4, 128, _D), jnp.float32),
        pltpu.VMEM_SHARED((_NPAD, _D), jnp.float32),
    ]
    + [pltpu.SemaphoreType.DMA] * 10,
)
def _prop(
    table, srcg, dstg, valf, p0, p1, src_v, dst_v, val_v, rows_v, acc,
    g0, g1, g2, g3, s0, s1, s2, s3, stsem, zsem,
):
    cid = lax.axis_index("c")
    sid = lax.axis_index("s")
    wid = sid * 2 + cid
    gsem = [g0, g1, g2, g3]
    ssem = [s0, s1, s2, s3]

    def _scat_wait(b):
        pltpu.make_async_copy(
            table.at[pl.ds(0, 128)], rows_v.at[b], ssem[b]
        ).wait()

    def _zrow(i, carry):
        z = jnp.zeros((16,), jnp.float32)
        rows_v[0, i, pl.ds(0, 16)] = z
        rows_v[0, i, pl.ds(16, 16)] = z
        return carry

    lax.fori_loop(0, 128, _zrow, 0)
    zdescs = [
        pltpu.async_copy(
            rows_v.at[0], acc.at[pl.ds(sid * _RPT + h * 128, 128)], zsem
        )
        for h in range(_RPT // 128)
    ]
    for dsc in zdescs:
        dsc.wait()
    plsc.subcore_barrier()

    row00 = wid * _GPW
    pltpu.async_copy(srcg.at[pl.ds(row00, _K)], src_v.at[0], stsem)
    pltpu.async_copy(dstg.at[pl.ds(row00, _K)], dst_v.at[0], stsem)
    pltpu.async_copy(valf.at[pl.ds(row00 * 128, _C)], val_v.at[0], stsem)

    def _scale(b, voff, val_p):
        @plsc.parallel_loop(0, 8, unroll=2)
        def _sg(g):
            v16 = val_p[pl.ds(voff + g * 16, 16)]
            for i in range(16):
                e = g * 16 + i
                sp = lax.gather(
                    v16,
                    jnp.full((16, 1), i, jnp.int32),
                    lax.GatherDimensionNumbers(
                        offset_dims=(),
                        collapsed_slice_dims=(0,),
                        start_index_map=(0,),
                    ),
                    (1,),
                    mode=lax.GatherScatterMode.PROMISE_IN_BOUNDS,
                )
                rows_v[b, e, pl.ds(0, 16)] = rows_v[b, e, pl.ds(0, 16)] * sp
                rows_v[b, e, pl.ds(16, 16)] = rows_v[b, e, pl.ds(16, 16)] * sp

    def _chunk(c, carry):
        p = lax.rem(c, 2)
        pltpu.make_async_copy(srcg.at[pl.ds(0, _K)], src_v.at[p], stsem).wait()
        pltpu.make_async_copy(dstg.at[pl.ds(0, _K)], dst_v.at[p], stsem).wait()
        pltpu.make_async_copy(valf.at[pl.ds(0, _C)], val_v.at[p], stsem).wait()
        @pl.when(c < _CHUNKS - 1)
        def _():
            cn = c + 1
            pn = lax.rem(cn, 2)
            rown = wid * _GPW + cn * _K
            pltpu.async_copy(srcg.at[pl.ds(rown, _K)], src_v.at[pn], stsem)
            pltpu.async_copy(dstg.at[pl.ds(rown, _K)], dst_v.at[pn], stsem)
            pltpu.async_copy(valf.at[pl.ds(rown * 128, _C)], val_v.at[pn], stsem)

        @pl.when(c > 0)
        def _():
            _scat_wait(0)
            _scat_wait(1)

        gdescs = [None] * _K
        sdescs = [None] * _K
        for j in range(2):
            gdescs[j] = pltpu.async_copy(
                table.at[src_v.at[p, j]], rows_v.at[j], gsem[j]
            )
        for j in range(_K):
            b = j % 4
            if j < _K - 2:
                b2 = (j + 2) % 4
                if j + 2 < 4:

                    @pl.when(c > 0)
                    def _():
                        _scat_wait(b2)

                else:
                    sdescs[j - 2].wait()
                gdescs[j + 2] = pltpu.async_copy(
                    table.at[src_v.at[p, j + 2]], rows_v.at[b2], gsem[b2]
                )
            gdescs[j].wait()
            _scale(b, j * 128, val_v.at[p])
            sdescs[j] = pltpu.async_copy(
                rows_v.at[b], acc.at[dst_v.at[p, j]], ssem[b], add=True
            )
        return carry

    lax.fori_loop(0, _CHUNKS, _chunk, 0)
    for b in range(4):
        _scat_wait(b)

    plsc.subcore_barrier()
    r0 = sid * _RPT

    @pl.when(cid == 0)
    def _():
        pltpu.sync_copy(acc.at[pl.ds(r0, _RPT)], p0.at[pl.ds(r0, _RPT)])

    @pl.when(cid == 1)
    def _():
        pltpu.sync_copy(acc.at[pl.ds(r0, _RPT)], p1.at[pl.ds(r0, _RPT)])


def _merge_body(p0_ref, p1_ref, s_ref, t_out, s_out):
    t = p0_ref[...] + p1_ref[...]
    t_out[...] = t
    s_out[...] = s_ref[...] + t


def _merge(p0, p1, s):
    rows = _NPAD * _D // 128
    blk = rows // 8
    f = pl.pallas_call(
        _merge_body,
        out_shape=[jax.ShapeDtypeStruct((rows, 128), jnp.float32)] * 2,
        grid=(8,),
        in_specs=[pl.BlockSpec((blk, 128), lambda i: (i, 0))] * 3,
        out_specs=[pl.BlockSpec((blk, 128), lambda i: (i, 0))] * 2,
    )
    t, s2 = f(
        p0.reshape(rows, 128), p1.reshape(rows, 128), s.reshape(rows, 128)
    )
    return t.reshape(_NPAD, _D), s2.reshape(_NPAD, _D)


@functools.partial(
    pl.kernel,
    out_type=[
        jax.ShapeDtypeStruct((4096, _D), jnp.float32),
        jax.ShapeDtypeStruct((4096, _D), jnp.float32),
    ],
    mesh=_MESH,
    compiler_params=pltpu.CompilerParams(use_tc_tiling_on_sc=False),
    scratch_types=[
        pltpu.VMEM((128,), jnp.int32),
        pltpu.VMEM((128,), jnp.int32),
        pltpu.VMEM((128, _D), jnp.float32),
        pltpu.VMEM((128, _D), jnp.float32),
        pltpu.SemaphoreType.DMA,
    ],
)
def _gather2(sum_t, uid, gid, ur_o, ir_o, uid_v, gid_v, ur_v, ir_v, sem):
    cid = lax.axis_index("c")
    sid = lax.axis_index("s")
    base = (sid * 2 + cid) * 128
    pltpu.sync_copy(uid.at[pl.ds(base, 128)], uid_v)
    pltpu.sync_copy(gid.at[pl.ds(base, 128)], gid_v)
    a = pltpu.async_copy(sum_t.at[uid_v], ur_v, sem)
    b = pltpu.async_copy(sum_t.at[gid_v], ir_v, sem)
    a.wait()
    b.wait()
    pltpu.sync_copy(ur_v, ur_o.at[pl.ds(base, 128)])
    pltpu.sync_copy(ir_v, ir_o.at[pl.ds(base, 128)])


def _dot_body(u_ref, i_ref, o_ref):
    o_ref[...] = jnp.sum(u_ref[...] * i_ref[...], axis=1) * jnp.float32(1.0 / 16.0)


def kernel(users, items, user_emb, item_emb, edge_src, edge_dst, edge_val):
    table0 = (
        jnp.zeros((_NPAD, _D), jnp.float32)
        .at[:_N]
        .set(jnp.concatenate([user_emb, item_emb], axis=0))
    )
    pad = _EPAD - _E
    srcg = jnp.concatenate([edge_src, jnp.zeros((pad,), jnp.int32)]).reshape(
        _EPAD // 128, 128
    )
    dstg = jnp.concatenate([edge_dst, jnp.zeros((pad,), jnp.int32)]).reshape(
        _EPAD // 128, 128
    )
    valf = jnp.concatenate([edge_val, jnp.zeros((pad,), jnp.float32)])
    table = table0
    s = table0
    for _ in range(3):
        p0, p1 = _prop(table, srcg, dstg, valf)
        table, s = _merge(p0, p1, s)
    ur, ir = _gather2(s, users, items + jnp.int32(_N_USERS))
    return pl.pallas_call(
        _dot_body, out_shape=jax.ShapeDtypeStruct((4096,), jnp.float32)
    )(ur, ir)
